# Initial kernel scaffold; baseline (speedup 1.0000x reference)
import functools

import jax
import jax.numpy as jnp
from jax import lax
from jax.experimental import pallas as pl
from jax.experimental.pallas import tpu as pltpu
from jax.experimental.pallas import tpu_sc as plsc

VOCAB = 100000
DIM = 64
NEG_K = 10
BATCH = 16384
CDF_LEN = 1024

NC, NS, L = 2, 16, 16
NW = NC * NS
RPW = BATCH // NW
SPW = RPW * NEG_K


def _cdf_body(counts_ref, cdf_ref):
    c = counts_ref[...]
    p = jnp.exp(0.75 * jnp.log(jnp.maximum(c, 1e-30)))
    p = jnp.where(c > 0.0, p, 0.0)
    p = p / jnp.sum(p)
    r = lax.broadcasted_iota(jnp.int32, (128, 128), 0)
    col = lax.broadcasted_iota(jnp.int32, (128, 128), 1)
    tri = (r <= col).astype(jnp.float32)
    rowcum = jnp.dot(p, tri, preferred_element_type=jnp.float32)
    rowsum = jnp.sum(p, axis=1, keepdims=True)
    ri = lax.broadcasted_iota(jnp.int32, (8, 8), 0)
    ci = lax.broadcasted_iota(jnp.int32, (8, 8), 1)
    strict = (ci < ri).astype(jnp.float32)
    off = jnp.dot(strict, rowsum, preferred_element_type=jnp.float32)
    cdf_ref[...] = rowcum + off


def _hash_u32(x):
    x = x ^ (x >> jnp.uint32(16))
    x = x * jnp.uint32(0x85EBCA6B)
    x = x ^ (x >> jnp.uint32(13))
    x = x * jnp.uint32(0xC2B2AE35)
    x = x ^ (x >> jnp.uint32(16))
    return x


def _sc_body(centers2d, pos2d, cdf_hbm, ein_hbm, eout_hbm, pp_hbm, np_hbm,
             cdf_v, cidx_v, pidx_v, negidx_v, vc_v, uo_v, uk_v, part_v, sem):
    cid = lax.axis_index("c")
    sid = lax.axis_index("s")
    wid = sid * NC + cid
    base = wid * RPW
    base4 = wid * 4

    pltpu.sync_copy(cdf_hbm, cdf_v)
    pltpu.sync_copy(centers2d.at[pl.ds(base4, 4)], cidx_v)
    pltpu.sync_copy(pos2d.at[pl.ds(base4, 4)], pidx_v)

    copies = []
    for j in range(4):
        copies.append(pltpu.async_copy(
            ein_hbm.at[cidx_v.at[j]], vc_v.at[pl.ds(j * 128, 128)], sem))
        copies.append(pltpu.async_copy(
            eout_hbm.at[pidx_v.at[j]], uo_v.at[pl.ds(j * 128, 128)], sem))

    lanes = lax.iota(jnp.int32, L)

    def sample_row(i, _):
        def sample_vec(j, _):
            ctr = (wid * SPW + i * 128 + j * L + lanes).astype(jnp.uint32)
            h = _hash_u32(ctr)
            frac = plsc.bitcast(h & jnp.uint32(0xFFFFFF), jnp.int32)
            u = frac.astype(jnp.float32) * (1.0 / 16777216.0)
            cnt = jnp.zeros((L,), jnp.int32)
            for step in (512, 256, 128, 64, 32, 16, 8, 4, 2, 1):
                m = cnt + step
                vals = plsc.load_gather(cdf_v, [m - 1])
                cnt = jnp.where(vals <= u, m, cnt)
            negidx_v[i, pl.ds(j * L, L)] = jnp.minimum(cnt, VOCAB - 1)
            return 0

        return lax.fori_loop(0, 128 // L, sample_vec, 0)

    lax.fori_loop(0, SPW // 128, sample_row, 0)

    for c in copies:
        c.wait()

    def pos_row(r, _):
        acc = vc_v[r, pl.ds(0, L)] * uo_v[r, pl.ds(0, L)]
        for d in range(1, DIM // L):
            acc = acc + vc_v[r, pl.ds(d * L, L)] * uo_v[r, pl.ds(d * L, L)]
        part_v[r, :] = acc
        return 0

    lax.fori_loop(0, RPW, pos_row, 0)
    pltpu.sync_copy(part_v, pp_hbm.at[pl.ds(base, RPW)])

    def neg_row(r, _):
        acc = uk_v[r, pl.ds(0, L)] * vc_v[r, pl.ds(0, L)]
        for d in range(1, DIM // L):
            acc = acc + uk_v[r, pl.ds(d * L, L)] * vc_v[r, pl.ds(d * L, L)]
        part_v[r, :] = acc
        return 0

    for k in range(NEG_K):
        kcopies = [
            pltpu.async_copy(
                eout_hbm.at[negidx_v.at[k * 4 + j]],
                uk_v.at[pl.ds(j * 128, 128)], sem)
            for j in range(4)
        ]
        for c in kcopies:
            c.wait()
        lax.fori_loop(0, RPW, neg_row, 0)
        pltpu.sync_copy(part_v, np_hbm.at[pl.ds((wid * NEG_K + k) * RPW, RPW)])


_sgns_sc = functools.partial(
    pl.kernel,
    out_type=[
        jax.ShapeDtypeStruct((BATCH, L), jnp.float32),
        jax.ShapeDtypeStruct((BATCH * NEG_K, L), jnp.float32),
    ],
    mesh=plsc.VectorSubcoreMesh(core_axis_name="c", subcore_axis_name="s"),
    scratch_types=[
        pltpu.VMEM((CDF_LEN,), jnp.float32),
        pltpu.VMEM((4, 128), jnp.int32),
        pltpu.VMEM((4, 128), jnp.int32),
        pltpu.VMEM((SPW // 128, 128), jnp.int32),
        pltpu.VMEM((RPW, DIM), jnp.float32),
        pltpu.VMEM((RPW, DIM), jnp.float32),
        pltpu.VMEM((RPW, DIM), jnp.float32),
        pltpu.VMEM((RPW, L), jnp.float32),
        pltpu.SemaphoreType.DMA,
    ],
)(_sc_body)


def _loss_body(pp_ref, np_ref, out_ref):
    i = pl.program_id(0)
    n = pl.num_programs(0)

    def logsig(x):
        return jnp.minimum(x, 0.0) - jnp.log(1.0 + jnp.exp(-jnp.abs(x)))

    ps = jnp.sum(pp_ref[...], axis=1)
    ns = jnp.sum(np_ref[...], axis=1)
    partial = -jnp.sum(logsig(ps)) - jnp.sum(logsig(-ns))
    acc = jnp.where(i == 0, 0.0, out_ref[0, 0]) + partial
    out_ref[0, 0] = jnp.where(i == n - 1, acc / BATCH, acc)


@jax.jit
def kernel(centers, pos, embed_in, embed_out, counts):
    counts_p = jnp.pad(counts.astype(jnp.float32),
                       (0, CDF_LEN - counts.shape[0])).reshape(8, 128)
    cdf8 = pl.pallas_call(
        _cdf_body,
        out_shape=jax.ShapeDtypeStruct((8, 128), jnp.float32),
    )(counts_p)
    cdf = cdf8.reshape(CDF_LEN)

    centers2d = centers.astype(jnp.int32).reshape(128, 128)
    pos2d = pos.astype(jnp.int32).reshape(128, 128)
    pp, npart = _sgns_sc(centers2d, pos2d, cdf, embed_in, embed_out)

    grid = 16
    loss = pl.pallas_call(
        _loss_body,
        grid=(grid,),
        in_specs=[
            pl.BlockSpec((BATCH // grid, L), lambda i: (i, 0)),
            pl.BlockSpec((BATCH * NEG_K // grid, L), lambda i: (i, 0)),
        ],
        out_specs=pl.BlockSpec(
            block_shape=(1, 1), index_map=lambda i: (0, 0),
            memory_space=pltpu.SMEM),
        out_shape=jax.ShapeDtypeStruct((1, 1), jnp.float32),
    )(pp, npart)
    return loss[0, 0]


# speedup vs baseline: 9.6816x; 9.6816x over previous
import functools

import jax
import jax.numpy as jnp
from jax import lax
from jax.experimental import pallas as pl
from jax.experimental.pallas import tpu as pltpu
from jax.experimental.pallas import tpu_sc as plsc

VOCAB = 100000
DIM = 64
NEG_K = 10
BATCH = 16384
CDF_LEN = 1024

NC, NS, L = 2, 16, 16
NW = NC * NS
RPW = BATCH // NW
SPW = RPW * NEG_K


def _cdf_body(counts_ref, cdf_ref):
    c = counts_ref[...]
    p = jnp.exp(0.75 * jnp.log(jnp.maximum(c, 1e-30)))
    p = jnp.where(c > 0.0, p, 0.0)
    p = p / jnp.sum(p)
    r = lax.broadcasted_iota(jnp.int32, (128, 128), 0)
    col = lax.broadcasted_iota(jnp.int32, (128, 128), 1)
    tri = (r <= col).astype(jnp.float32)
    rowcum = jnp.dot(p, tri, preferred_element_type=jnp.float32)
    rowsum = jnp.sum(p, axis=1, keepdims=True)
    ri = lax.broadcasted_iota(jnp.int32, (8, 8), 0)
    ci = lax.broadcasted_iota(jnp.int32, (8, 8), 1)
    strict = (ci < ri).astype(jnp.float32)
    off = jnp.dot(strict, rowsum, preferred_element_type=jnp.float32)
    cdf_ref[...] = rowcum + off


def _hash_u32(x):
    x = x ^ (x >> jnp.uint32(16))
    x = x * jnp.uint32(0x85EBCA6B)
    x = x ^ (x >> jnp.uint32(13))
    x = x * jnp.uint32(0xC2B2AE35)
    x = x ^ (x >> jnp.uint32(16))
    return x


def _sc_body(centers2d, pos2d, cdf_hbm, ein_hbm, eout_hbm, pp_hbm, np_hbm,
             cdf_v, cidx_v, pidx_v, negidx_v, vc_v, uo_v, uk_v, part_v, sem):
    cid = lax.axis_index("c")
    sid = lax.axis_index("s")
    wid = sid * NC + cid
    base = wid * RPW
    base4 = wid * 4

    pltpu.sync_copy(cdf_hbm, cdf_v)
    pltpu.sync_copy(centers2d.at[pl.ds(base4, 4)], cidx_v)
    pltpu.sync_copy(pos2d.at[pl.ds(base4, 4)], pidx_v)

    copies = []
    for j in range(4):
        copies.append(pltpu.async_copy(
            ein_hbm.at[cidx_v.at[j]], vc_v.at[pl.ds(j * 128, 128)], sem))
        copies.append(pltpu.async_copy(
            eout_hbm.at[pidx_v.at[j]], uo_v.at[pl.ds(j * 128, 128)], sem))

    lanes = lax.iota(jnp.int32, L)

    def sample_row(i, _):
        def sample_vec(j, _):
            ctr = (wid * SPW + i * 128 + j * L + lanes).astype(jnp.uint32)
            h = _hash_u32(ctr)
            frac = plsc.bitcast(h & jnp.uint32(0xFFFFFF), jnp.int32)
            u = frac.astype(jnp.float32) * (1.0 / 16777216.0)
            cnt = jnp.zeros((L,), jnp.int32)
            for step in (512, 256, 128, 64, 32, 16, 8, 4, 2, 1):
                m = cnt + step
                vals = plsc.load_gather(cdf_v, [m - 1])
                cnt = jnp.where(vals <= u, m, cnt)
            negidx_v[i, pl.ds(j * L, L)] = jnp.minimum(cnt, VOCAB - 1)
            return 0

        return lax.fori_loop(0, 128 // L, sample_vec, 0)

    lax.fori_loop(0, SPW // 128, sample_row, 0)

    for c in copies:
        c.wait()

    def pos_row(r, _):
        acc = vc_v[r, pl.ds(0, L)] * uo_v[r, pl.ds(0, L)]
        for d in range(1, DIM // L):
            acc = acc + vc_v[r, pl.ds(d * L, L)] * uo_v[r, pl.ds(d * L, L)]
        part_v[r, :] = acc
        return 0

    lax.fori_loop(0, RPW, pos_row, 0)
    pltpu.sync_copy(part_v, pp_hbm.at[pl.ds(base, RPW)])

    def neg_row(r, _):
        acc = uk_v[r, pl.ds(0, L)] * vc_v[r, pl.ds(0, L)]
        for d in range(1, DIM // L):
            acc = acc + uk_v[r, pl.ds(d * L, L)] * vc_v[r, pl.ds(d * L, L)]
        part_v[r, :] = acc
        return 0

    for k in range(NEG_K):
        kcopies = [
            pltpu.async_copy(
                eout_hbm.at[negidx_v.at[k * 4 + j]],
                uk_v.at[pl.ds(j * 128, 128)], sem)
            for j in range(4)
        ]
        for c in kcopies:
            c.wait()
        lax.fori_loop(0, RPW, neg_row, 0)
        pltpu.sync_copy(part_v, np_hbm.at[pl.ds((wid * NEG_K + k) * RPW, RPW)])


_sgns_sc = functools.partial(
    pl.kernel,
    out_type=[
        jax.ShapeDtypeStruct((BATCH, L), jnp.float32),
        jax.ShapeDtypeStruct((BATCH * NEG_K, L), jnp.float32),
    ],
    mesh=plsc.VectorSubcoreMesh(core_axis_name="c", subcore_axis_name="s"),
    scratch_types=[
        pltpu.VMEM((CDF_LEN,), jnp.float32),
        pltpu.VMEM((4, 128), jnp.int32),
        pltpu.VMEM((4, 128), jnp.int32),
        pltpu.VMEM((SPW // 128, 128), jnp.int32),
        pltpu.VMEM((RPW, DIM), jnp.float32),
        pltpu.VMEM((RPW, DIM), jnp.float32),
        pltpu.VMEM((RPW, DIM), jnp.float32),
        pltpu.VMEM((RPW, L), jnp.float32),
        pltpu.SemaphoreType.DMA,
    ],
    compiler_params=pltpu.CompilerParams(
        needs_layout_passes=False, use_tc_tiling_on_sc=False),
)(_sc_body)


def _loss_body(pp_ref, np_ref, out_ref):
    i = pl.program_id(0)
    n = pl.num_programs(0)

    def logsig(x):
        return jnp.minimum(x, 0.0) - jnp.log(1.0 + jnp.exp(-jnp.abs(x)))

    ps = jnp.sum(pp_ref[...], axis=1)
    ns = jnp.sum(np_ref[...], axis=1)
    partial = -jnp.sum(logsig(ps)) - jnp.sum(logsig(-ns))
    acc = jnp.where(i == 0, 0.0, out_ref[0, 0]) + partial
    out_ref[0, 0] = jnp.where(i == n - 1, acc / BATCH, acc)


@jax.jit
def kernel(centers, pos, embed_in, embed_out, counts):
    counts_p = jnp.pad(counts.astype(jnp.float32),
                       (0, CDF_LEN - counts.shape[0])).reshape(8, 128)
    cdf8 = pl.pallas_call(
        _cdf_body,
        out_shape=jax.ShapeDtypeStruct((8, 128), jnp.float32),
    )(counts_p)
    cdf = cdf8.reshape(CDF_LEN)

    centers2d = centers.astype(jnp.int32).reshape(128, 128)
    pos2d = pos.astype(jnp.int32).reshape(128, 128)
    pp, npart = _sgns_sc(centers2d, pos2d, cdf, embed_in, embed_out)

    grid = 16
    loss = pl.pallas_call(
        _loss_body,
        grid=(grid,),
        in_specs=[
            pl.BlockSpec((BATCH // grid, L), lambda i: (i, 0)),
            pl.BlockSpec((BATCH * NEG_K // grid, L), lambda i: (i, 0)),
        ],
        out_specs=pl.BlockSpec(
            block_shape=(1, 1), index_map=lambda i: (0, 0),
            memory_space=pltpu.SMEM),
        out_shape=jax.ShapeDtypeStruct((1, 1), jnp.float32),
    )(pp, npart)
    return loss[0, 0]


# speedup vs baseline: 13.9023x; 1.4360x over previous
import functools

import jax
import jax.numpy as jnp
from jax import lax
from jax.experimental import pallas as pl
from jax.experimental.pallas import tpu as pltpu
from jax.experimental.pallas import tpu_sc as plsc

VOCAB = 100000
DIM = 64
NEG_K = 10
BATCH = 16384
CDF_LEN = 1024

NC, NS, L = 2, 16, 16
NW = NC * NS
RPW = BATCH // NW
SPW = RPW * NEG_K
NB = 8
BR = RPW // NB
GPR = 128 // L

PP_ROWS = BATCH // GPR
NP_ROWS = BATCH * NEG_K // GPR


def _cdf_body(counts_ref, cdf_ref):
    c = counts_ref[...]
    p = jnp.exp(0.75 * jnp.log(jnp.maximum(c, 1e-30)))
    p = jnp.where(c > 0.0, p, 0.0)
    p = p / jnp.sum(p)
    r = lax.broadcasted_iota(jnp.int32, (128, 128), 0)
    col = lax.broadcasted_iota(jnp.int32, (128, 128), 1)
    tri = (r <= col).astype(jnp.float32)
    rowcum = jnp.dot(p, tri, preferred_element_type=jnp.float32)
    rowsum = jnp.sum(p, axis=1, keepdims=True)
    ri = lax.broadcasted_iota(jnp.int32, (8, 8), 0)
    ci = lax.broadcasted_iota(jnp.int32, (8, 8), 1)
    strict = (ci < ri).astype(jnp.float32)
    off = jnp.dot(strict, rowsum, preferred_element_type=jnp.float32)
    cdf_ref[...] = rowcum + off


def _hash_u32(x):
    x = x ^ (x >> jnp.uint32(16))
    x = x * jnp.uint32(0x85EBCA6B)
    x = x ^ (x >> jnp.uint32(13))
    x = x * jnp.uint32(0xC2B2AE35)
    x = x ^ (x >> jnp.uint32(16))
    return x


def _sc_body(centers_hbm, pos_hbm, cdf_hbm, ein_hbm, eout_hbm, pp_hbm, np_hbm,
             cdf_v, cidx_v, pidx_v, negidx_v, vc_v, uo_v, uk_v, pp_v, np_v,
             sem0, sem1):
    cid = lax.axis_index("c")
    sid = lax.axis_index("s")
    wid = sid * NC + cid
    base = wid * RPW
    sems = (sem0, sem1)

    pltpu.sync_copy(cdf_hbm, cdf_v)
    pltpu.sync_copy(centers_hbm.at[pl.ds(base, RPW)], cidx_v)
    pltpu.sync_copy(pos_hbm.at[pl.ds(base, RPW)], pidx_v)

    def issue_vcuo(b, buf):
        return [
            pltpu.async_copy(ein_hbm.at[cidx_v.at[pl.ds(b * BR, BR)]],
                             vc_v.at[buf], sems[buf]),
            pltpu.async_copy(eout_hbm.at[pidx_v.at[pl.ds(b * BR, BR)]],
                             uo_v.at[buf], sems[buf]),
        ]

    def issue_uk(b, buf):
        return [
            pltpu.async_copy(
                eout_hbm.at[negidx_v.at[pl.ds((b * NEG_K + k) * BR, BR)]],
                uk_v.at[buf, pl.ds(k * BR, BR)], sems[buf])
            for k in range(NEG_K)
        ]

    pend = [issue_vcuo(0, 0), issue_vcuo(1, 1)]

    lanes = lax.iota(jnp.int32, L)

    def sample_vec(t, _):
        ctr = (wid * SPW + t * L + lanes).astype(jnp.uint32)
        h = _hash_u32(ctr)
        frac = plsc.bitcast(h & jnp.uint32(0xFFFFFF), jnp.int32)
        u = frac.astype(jnp.float32) * (1.0 / 16777216.0)
        cnt = jnp.zeros((L,), jnp.int32)
        for step in (512, 256, 128, 64, 32, 16, 8, 4, 2, 1):
            m = cnt + step
            vals = plsc.load_gather(cdf_v, [m - 1])
            cnt = jnp.where(vals <= u, m, cnt)
        negidx_v[pl.ds(t * L, L)] = jnp.minimum(cnt, VOCAB - 1)
        return 0

    lax.fori_loop(0, SPW // L, sample_vec, 0)

    pend[0] += issue_uk(0, 0)
    pend[1] += issue_uk(1, 1)

    for b in range(NB):
        buf = b % 2
        for c in pend[b]:
            c.wait()
        if b + 2 < NB:
            pend.append(issue_vcuo(b + 2, buf) + issue_uk(b + 2, buf))

        def row_body(r, _):
            v0 = vc_v[buf, r, pl.ds(0 * L, L)]
            v1 = vc_v[buf, r, pl.ds(1 * L, L)]
            v2 = vc_v[buf, r, pl.ds(2 * L, L)]
            v3 = vc_v[buf, r, pl.ds(3 * L, L)]
            pr = r // GPR
            lo = (r % GPR) * L
            acc = (v0 * uo_v[buf, r, pl.ds(0 * L, L)]
                   + v1 * uo_v[buf, r, pl.ds(1 * L, L)]
                   + v2 * uo_v[buf, r, pl.ds(2 * L, L)]
                   + v3 * uo_v[buf, r, pl.ds(3 * L, L)])
            pp_v[pr, pl.ds(lo, L)] = acc
            for k in range(NEG_K):
                kr = k * BR + r
                a = (v0 * uk_v[buf, kr, pl.ds(0 * L, L)]
                     + v1 * uk_v[buf, kr, pl.ds(1 * L, L)]
                     + v2 * uk_v[buf, kr, pl.ds(2 * L, L)]
                     + v3 * uk_v[buf, kr, pl.ds(3 * L, L)])
                np_v[k * (BR // GPR) + pr, pl.ds(lo, L)] = a
            return 0

        lax.fori_loop(0, BR, row_body, 0)

        pltpu.sync_copy(
            pp_v, pp_hbm.at[pl.ds(wid * (RPW // GPR) + b * (BR // GPR),
                                  BR // GPR)])
        pltpu.sync_copy(
            np_v, np_hbm.at[pl.ds((wid * NB + b) * (BR * NEG_K // GPR),
                                  BR * NEG_K // GPR)])


_sgns_sc = functools.partial(
    pl.kernel,
    out_type=[
        jax.ShapeDtypeStruct((PP_ROWS, 128), jnp.float32),
        jax.ShapeDtypeStruct((NP_ROWS, 128), jnp.float32),
    ],
    mesh=plsc.VectorSubcoreMesh(core_axis_name="c", subcore_axis_name="s"),
    scratch_types=[
        pltpu.VMEM((CDF_LEN,), jnp.float32),
        pltpu.VMEM((RPW,), jnp.int32),
        pltpu.VMEM((RPW,), jnp.int32),
        pltpu.VMEM((SPW,), jnp.int32),
        pltpu.VMEM((2, BR, DIM), jnp.float32),
        pltpu.VMEM((2, BR, DIM), jnp.float32),
        pltpu.VMEM((2, BR * NEG_K, DIM), jnp.float32),
        pltpu.VMEM((BR // GPR, 128), jnp.float32),
        pltpu.VMEM((BR * NEG_K // GPR, 128), jnp.float32),
        pltpu.SemaphoreType.DMA,
        pltpu.SemaphoreType.DMA,
    ],
    compiler_params=pltpu.CompilerParams(
        needs_layout_passes=False, use_tc_tiling_on_sc=False),
)(_sc_body)


def _loss_body(pp_ref, np_ref, out_ref):
    i = pl.program_id(0)
    n = pl.num_programs(0)
    lane = lax.broadcasted_iota(jnp.int32, (128, GPR), 0)
    grp = lax.broadcasted_iota(jnp.int32, (128, GPR), 1)
    gmat = (lane // L == grp).astype(jnp.float32)

    def logsig(x):
        return jnp.minimum(x, 0.0) - jnp.log(1.0 + jnp.exp(-jnp.abs(x)))

    ps = jnp.dot(pp_ref[...], gmat, preferred_element_type=jnp.float32)
    ns = jnp.dot(np_ref[...], gmat, preferred_element_type=jnp.float32)
    partial = -jnp.sum(logsig(ps)) - jnp.sum(logsig(-ns))
    acc = jnp.where(i == 0, 0.0, out_ref[0, 0]) + partial
    out_ref[0, 0] = jnp.where(i == n - 1, acc / BATCH, acc)


@jax.jit
def kernel(centers, pos, embed_in, embed_out, counts):
    counts_p = jnp.pad(counts.astype(jnp.float32),
                       (0, CDF_LEN - counts.shape[0])).reshape(8, 128)
    cdf8 = pl.pallas_call(
        _cdf_body,
        out_shape=jax.ShapeDtypeStruct((8, 128), jnp.float32),
    )(counts_p)
    cdf = cdf8.reshape(CDF_LEN)

    pp, npart = _sgns_sc(centers.astype(jnp.int32), pos.astype(jnp.int32),
                         cdf, embed_in, embed_out)

    grid = 16
    loss = pl.pallas_call(
        _loss_body,
        grid=(grid,),
        in_specs=[
            pl.BlockSpec((PP_ROWS // grid, 128), lambda i: (i, 0)),
            pl.BlockSpec((NP_ROWS // grid, 128), lambda i: (i, 0)),
        ],
        out_specs=pl.BlockSpec(
            block_shape=(1, 1), index_map=lambda i: (0, 0),
            memory_space=pltpu.SMEM),
        out_shape=jax.ShapeDtypeStruct((1, 1), jnp.float32),
    )(pp, npart)
    return loss[0, 0]


# speedup vs baseline: 15.9740x; 1.1490x over previous
import functools

import jax
import jax.numpy as jnp
from jax import lax
from jax.experimental import pallas as pl
from jax.experimental.pallas import tpu as pltpu
from jax.experimental.pallas import tpu_sc as plsc

VOCAB = 100000
DIM = 64
NEG_K = 10
BATCH = 16384
CDF_LEN = 1024

NC, NS, L = 2, 16, 16
NW = NC * NS
RPW = BATCH // NW
SPW = RPW * NEG_K
NB = 8
BR = RPW // NB
GPR = 128 // L

PP_ROWS = BATCH // GPR
NP_ROWS = BATCH * NEG_K // GPR


def _cdf_body(counts_ref, cdf_ref):
    c = counts_ref[...]
    p = jnp.exp(0.75 * jnp.log(jnp.maximum(c, 1e-30)))
    p = jnp.where(c > 0.0, p, 0.0)
    p = p / jnp.sum(p)
    r = lax.broadcasted_iota(jnp.int32, (128, 128), 0)
    col = lax.broadcasted_iota(jnp.int32, (128, 128), 1)
    tri = (r <= col).astype(jnp.float32)
    rowcum = jnp.dot(p, tri, preferred_element_type=jnp.float32)
    rowsum = jnp.sum(p, axis=1, keepdims=True)
    ri = lax.broadcasted_iota(jnp.int32, (8, 8), 0)
    ci = lax.broadcasted_iota(jnp.int32, (8, 8), 1)
    strict = (ci < ri).astype(jnp.float32)
    off = jnp.dot(strict, rowsum, preferred_element_type=jnp.float32)
    cdf_ref[...] = rowcum + off


def _hash_u32(x):
    x = x ^ (x >> jnp.uint32(16))
    x = x * jnp.uint32(0x85EBCA6B)
    x = x ^ (x >> jnp.uint32(13))
    x = x * jnp.uint32(0xC2B2AE35)
    x = x ^ (x >> jnp.uint32(16))
    return x


def _sc_body(centers_hbm, pos_hbm, cdf_hbm, ein_hbm, eout_hbm, pp_hbm, np_hbm,
             cdf_v, cidx_v, pidx_v, negidx_v, vc_v, uo_v, uk_v, pp_v, np_v,
             sem0, sem1):
    cid = lax.axis_index("c")
    sid = lax.axis_index("s")
    wid = sid * NC + cid
    base = wid * RPW
    sems = (sem0, sem1)

    pltpu.sync_copy(cdf_hbm, cdf_v)
    pltpu.sync_copy(centers_hbm.at[pl.ds(base, RPW)], cidx_v)
    pltpu.sync_copy(pos_hbm.at[pl.ds(base, RPW)], pidx_v)

    def issue_vcuo(b, buf):
        return [
            pltpu.async_copy(ein_hbm.at[cidx_v.at[pl.ds(b * BR, BR)]],
                             vc_v.at[buf], sems[buf]),
            pltpu.async_copy(eout_hbm.at[pidx_v.at[pl.ds(b * BR, BR)]],
                             uo_v.at[buf], sems[buf]),
        ]

    def issue_uk(b, buf):
        return [
            pltpu.async_copy(
                eout_hbm.at[negidx_v.at[pl.ds((b * NEG_K + k) * BR, BR)]],
                uk_v.at[buf, pl.ds(k * BR, BR)], sems[buf])
            for k in range(NEG_K)
        ]

    pend = [issue_vcuo(0, 0), issue_vcuo(1, 1)]

    lanes = lax.iota(jnp.int32, L)

    @plsc.parallel_loop(0, SPW // L, 1, unroll=4)
    def sample_vec(t):
        ctr = (wid * SPW + t * L + lanes).astype(jnp.uint32)
        h = _hash_u32(ctr)
        frac = plsc.bitcast(h & jnp.uint32(0xFFFFFF), jnp.int32)
        u = frac.astype(jnp.float32) * (1.0 / 16777216.0)
        cnt = jnp.zeros((L,), jnp.int32)
        for step in (512, 256, 128, 64, 32, 16, 8, 4, 2, 1):
            m = cnt + step
            vals = plsc.load_gather(cdf_v, [m - 1])
            cnt = jnp.where(vals <= u, m, cnt)
        negidx_v[pl.ds(t * L, L)] = jnp.minimum(cnt, VOCAB - 1)

    pend[0] += issue_uk(0, 0)
    pend[1] += issue_uk(1, 1)

    for b in range(NB):
        buf = b % 2
        for c in pend[b]:
            c.wait()
        if b + 2 < NB:
            pend.append(issue_vcuo(b + 2, buf) + issue_uk(b + 2, buf))

        @plsc.parallel_loop(0, BR, 1, unroll=2)
        def row_body(r):
            v0 = vc_v[buf, r, pl.ds(0 * L, L)]
            v1 = vc_v[buf, r, pl.ds(1 * L, L)]
            v2 = vc_v[buf, r, pl.ds(2 * L, L)]
            v3 = vc_v[buf, r, pl.ds(3 * L, L)]
            pr = r // GPR
            lo = (r % GPR) * L
            acc = (v0 * uo_v[buf, r, pl.ds(0 * L, L)]
                   + v1 * uo_v[buf, r, pl.ds(1 * L, L)]
                   + v2 * uo_v[buf, r, pl.ds(2 * L, L)]
                   + v3 * uo_v[buf, r, pl.ds(3 * L, L)])
            pp_v[pr, pl.ds(lo, L)] = acc
            for k in range(NEG_K):
                kr = k * BR + r
                a = (v0 * uk_v[buf, kr, pl.ds(0 * L, L)]
                     + v1 * uk_v[buf, kr, pl.ds(1 * L, L)]
                     + v2 * uk_v[buf, kr, pl.ds(2 * L, L)]
                     + v3 * uk_v[buf, kr, pl.ds(3 * L, L)])
                np_v[k * (BR // GPR) + pr, pl.ds(lo, L)] = a

        pltpu.sync_copy(
            pp_v, pp_hbm.at[pl.ds(wid * (RPW // GPR) + b * (BR // GPR),
                                  BR // GPR)])
        pltpu.sync_copy(
            np_v, np_hbm.at[pl.ds((wid * NB + b) * (BR * NEG_K // GPR),
                                  BR * NEG_K // GPR)])


_sgns_sc = functools.partial(
    pl.kernel,
    out_type=[
        jax.ShapeDtypeStruct((PP_ROWS, 128), jnp.float32),
        jax.ShapeDtypeStruct((NP_ROWS, 128), jnp.float32),
    ],
    mesh=plsc.VectorSubcoreMesh(core_axis_name="c", subcore_axis_name="s"),
    scratch_types=[
        pltpu.VMEM((CDF_LEN,), jnp.float32),
        pltpu.VMEM((RPW,), jnp.int32),
        pltpu.VMEM((RPW,), jnp.int32),
        pltpu.VMEM((SPW,), jnp.int32),
        pltpu.VMEM((2, BR, DIM), jnp.float32),
        pltpu.VMEM((2, BR, DIM), jnp.float32),
        pltpu.VMEM((2, BR * NEG_K, DIM), jnp.float32),
        pltpu.VMEM((BR // GPR, 128), jnp.float32),
        pltpu.VMEM((BR * NEG_K // GPR, 128), jnp.float32),
        pltpu.SemaphoreType.DMA,
        pltpu.SemaphoreType.DMA,
    ],
    compiler_params=pltpu.CompilerParams(
        needs_layout_passes=False, use_tc_tiling_on_sc=False),
)(_sc_body)


def _loss_body(pp_ref, np_ref, out_ref):
    i = pl.program_id(0)
    n = pl.num_programs(0)
    lane = lax.broadcasted_iota(jnp.int32, (128, GPR), 0)
    grp = lax.broadcasted_iota(jnp.int32, (128, GPR), 1)
    gmat = (lane // L == grp).astype(jnp.float32)

    def logsig(x):
        return jnp.minimum(x, 0.0) - jnp.log(1.0 + jnp.exp(-jnp.abs(x)))

    ps = jnp.dot(pp_ref[...], gmat, preferred_element_type=jnp.float32)
    ns = jnp.dot(np_ref[...], gmat, preferred_element_type=jnp.float32)
    partial = -jnp.sum(logsig(ps)) - jnp.sum(logsig(-ns))
    acc = jnp.where(i == 0, 0.0, out_ref[0, 0]) + partial
    out_ref[0, 0] = jnp.where(i == n - 1, acc / BATCH, acc)


@jax.jit
def kernel(centers, pos, embed_in, embed_out, counts):
    counts_p = jnp.pad(counts.astype(jnp.float32),
                       (0, CDF_LEN - counts.shape[0])).reshape(8, 128)
    cdf8 = pl.pallas_call(
        _cdf_body,
        out_shape=jax.ShapeDtypeStruct((8, 128), jnp.float32),
    )(counts_p)
    cdf = cdf8.reshape(CDF_LEN)

    pp, npart = _sgns_sc(centers.astype(jnp.int32), pos.astype(jnp.int32),
                         cdf, embed_in, embed_out)

    grid = 16
    loss = pl.pallas_call(
        _loss_body,
        grid=(grid,),
        in_specs=[
            pl.BlockSpec((PP_ROWS // grid, 128), lambda i: (i, 0)),
            pl.BlockSpec((NP_ROWS // grid, 128), lambda i: (i, 0)),
        ],
        out_specs=pl.BlockSpec(
            block_shape=(1, 1), index_map=lambda i: (0, 0),
            memory_space=pltpu.SMEM),
        out_shape=jax.ShapeDtypeStruct((1, 1), jnp.float32),
    )(pp, npart)
    return loss[0, 0]
